# baseline (speedup 1.0000x reference)
import functools

import jax
import jax.numpy as jnp
from jax import lax
from jax.experimental import pallas as pl
from jax.experimental.pallas import tpu as pltpu
from jax.experimental.pallas import tpu_sc as plsc

N = 10000
E = 160000
D = 256
HALF = D // 2
NC = 2
NS = 16
CHUNK = 128
RPT = 640
NPAD = NS * RPT
UNROLL_B = 4
UNROLL_S = 2
EPAD = 163840


@functools.cache
def _sc_kernels():
    mesh = plsc.VectorSubcoreMesh(core_axis_name="c", subcore_axis_name="s")

    @functools.partial(
        pl.kernel,
        out_type=jax.ShapeDtypeStruct((NC, NPAD, 16), jnp.float32),
        mesh=mesh,
        scratch_types=[
            pltpu.VMEM((UNROLL_B, CHUNK), jnp.int32),
            pltpu.VMEM((CHUNK, 16), jnp.float32),
            pltpu.VMEM_SHARED((NPAD, 16), jnp.float32),
            pltpu.SemaphoreType.DMA((UNROLL_B,)),
        ],
    )
    def sc_bincount(src_hbm, zeros_hbm, ones_hbm, out_hbm, idxv, onesv, acc,
                    sem):
        c = lax.axis_index("c")
        s = lax.axis_index("s")
        row0 = s * RPT
        pltpu.sync_copy(zeros_hbm, acc.at[pl.ds(row0, RPT)])
        pltpu.sync_copy(ones_hbm, onesv)
        plsc.subcore_barrier()

        ehalf = EPAD // NC
        nch = ehalf // CHUNK
        iters = nch // (NS * UNROLL_B)

        @pl.loop(0, iters)
        def _(t):
            base = c * ehalf + (t * NS * UNROLL_B + s * UNROLL_B) * CHUNK
            for j in range(UNROLL_B):
                pltpu.async_copy(src_hbm.at[pl.ds(base + j * CHUNK, CHUNK)],
                                 idxv.at[j], sem.at[j])
            for j in range(UNROLL_B):
                pltpu.make_async_copy(src_hbm.at[pl.ds(base, CHUNK)],
                                      idxv.at[j], sem.at[j]).wait()
                pltpu.async_copy(onesv, acc.at[idxv.at[j]], sem.at[j],
                                 add=True)
            for j in range(UNROLL_B):
                pltpu.make_async_copy(onesv, acc.at[idxv.at[j]],
                                      sem.at[j]).wait()

        plsc.subcore_barrier()
        pltpu.sync_copy(acc.at[pl.ds(row0, RPT)],
                        out_hbm.at[c, pl.ds(row0, RPT)])

    @functools.partial(
        pl.kernel,
        out_type=jax.ShapeDtypeStruct((NC, NPAD, HALF), jnp.float32),
        mesh=mesh,
        scratch_types=[
            pltpu.VMEM((UNROLL_S, CHUNK), jnp.int32),
            pltpu.VMEM((UNROLL_S, CHUNK), jnp.int32),
            pltpu.VMEM((UNROLL_S, CHUNK), jnp.int32),
            pltpu.VMEM((UNROLL_S, CHUNK, HALF), jnp.float32),
            pltpu.VMEM_SHARED((NPAD, HALF), jnp.float32),
            pltpu.SemaphoreType.DMA((UNROLL_S,)),
            pltpu.SemaphoreType.DMA((UNROLL_S,)),
        ],
    )
    def sc_scatter(g_hbm, src_hbm, dst_hbm, zeros_hbm, out_hbm,
                   srcv, dstv, gidx, gbuf, acc, isem, gsem):
        c = lax.axis_index("c")
        s = lax.axis_index("s")
        row0 = s * RPT
        pltpu.sync_copy(zeros_hbm, acc.at[pl.ds(row0, RPT)])
        plsc.subcore_barrier()

        nch = EPAD // CHUNK
        iters = nch // (NS * UNROLL_S)

        @pl.loop(0, iters)
        def _(t):
            base = (t * NS * UNROLL_S + s * UNROLL_S) * CHUNK
            for j in range(UNROLL_S):
                pltpu.async_copy(src_hbm.at[pl.ds(base + j * CHUNK, CHUNK)],
                                 srcv.at[j], isem.at[j])
                pltpu.async_copy(dst_hbm.at[pl.ds(base + j * CHUNK, CHUNK)],
                                 dstv.at[j], isem.at[j])
            for j in range(UNROLL_S):
                pltpu.make_async_copy(src_hbm.at[pl.ds(base, CHUNK)],
                                      srcv.at[j], isem.at[j]).wait()
                pltpu.make_async_copy(dst_hbm.at[pl.ds(base, CHUNK)],
                                      dstv.at[j], isem.at[j]).wait()

                @pl.loop(0, CHUNK // 16)
                def _(i):
                    v = srcv[j, pl.ds(i * 16, 16)]
                    gidx[j, pl.ds(i * 16, 16)] = v * 2 + c

                pltpu.async_copy(g_hbm.at[gidx.at[j]], gbuf.at[j], gsem.at[j])
            for j in range(UNROLL_S):
                pltpu.make_async_copy(g_hbm.at[gidx.at[j]], gbuf.at[j],
                                      gsem.at[j]).wait()
                pltpu.async_copy(gbuf.at[j], acc.at[dstv.at[j]], gsem.at[j],
                                 add=True)
            for j in range(UNROLL_S):
                pltpu.make_async_copy(gbuf.at[j], acc.at[dstv.at[j]],
                                      gsem.at[j]).wait()

        plsc.subcore_barrier()
        pltpu.sync_copy(acc.at[pl.ds(row0, RPT)],
                        out_hbm.at[c, pl.ds(row0, RPT)])

    return sc_bincount, sc_scatter


_ROWS = 1000


def _tc_main_body(counts_ref, x_ref, w_ref, b_ref, root_ref, g_ref, self_ref):
    cnt = counts_ref[0, :, 0:1] + counts_ref[1, :, 0:1]
    degs = cnt + 1.0
    norm = lax.rsqrt(degs)
    h = jnp.dot(x_ref[...], w_ref[...],
                preferred_element_type=jnp.float32) + b_ref[...]
    g_ref[...] = norm * jnp.maximum(h, 0.0)
    self_ref[...] = jnp.maximum(h + root_ref[...], 0.0) / degs


def _tc_main(counts, x, w, b2, root):
    return pl.pallas_call(
        _tc_main_body,
        grid=(N // _ROWS,),
        in_specs=[
            pl.BlockSpec((NC, _ROWS, 16), lambda i: (0, i, 0)),
            pl.BlockSpec((_ROWS, D), lambda i: (i, 0)),
            pl.BlockSpec((D, D), lambda i: (0, 0)),
            pl.BlockSpec((1, D), lambda i: (0, 0)),
            pl.BlockSpec((1, D), lambda i: (0, 0)),
        ],
        out_specs=[
            pl.BlockSpec((_ROWS, D), lambda i: (i, 0)),
            pl.BlockSpec((_ROWS, D), lambda i: (i, 0)),
        ],
        out_shape=[
            jax.ShapeDtypeStruct((NPAD, D), jnp.float32),
            jax.ShapeDtypeStruct((N, D), jnp.float32),
        ],
    )(counts, x, w, b2, root)


def _tc_out_body(counts_ref, acc_ref, self_ref, o_ref):
    cnt = counts_ref[0, :, 0:1] + counts_ref[1, :, 0:1]
    norm = lax.rsqrt(cnt + 1.0)
    acc = jnp.concatenate([acc_ref[0], acc_ref[1]], axis=1)
    o_ref[...] = norm * acc + self_ref[...]


def _tc_out(counts, acc, self_term):
    return pl.pallas_call(
        _tc_out_body,
        grid=(N // _ROWS,),
        in_specs=[
            pl.BlockSpec((NC, _ROWS, 16), lambda i: (0, i, 0)),
            pl.BlockSpec((NC, _ROWS, HALF), lambda i: (0, i, 0)),
            pl.BlockSpec((_ROWS, D), lambda i: (i, 0)),
        ],
        out_specs=pl.BlockSpec((_ROWS, D), lambda i: (i, 0)),
        out_shape=jax.ShapeDtypeStruct((N, D), jnp.float32),
    )(counts, acc, self_term)


def kernel(x, edge_index, W, b, root_emb):
    sc_bincount, sc_scatter = _sc_kernels()
    pad = jnp.full((EPAD - E,), N, jnp.int32)
    src = jnp.concatenate([edge_index[0], pad])
    dst = jnp.concatenate([edge_index[1], pad])
    zeros16 = jnp.zeros((RPT, 16), jnp.float32)
    ones16 = jnp.ones((CHUNK, 16), jnp.float32)
    zeros128 = jnp.zeros((RPT, HALF), jnp.float32)

    counts = sc_bincount(src, zeros16, ones16)
    g, self_term = _tc_main(counts, x, W, b.reshape(1, D), root_emb)
    acc = sc_scatter(g.reshape(NC * NPAD, HALF), src, dst, zeros128)
    return _tc_out(counts, acc, self_term)


# speedup vs baseline: 1.0129x; 1.0129x over previous
import functools

import jax
import jax.numpy as jnp
from jax import lax
from jax.experimental import pallas as pl
from jax.experimental.pallas import tpu as pltpu
from jax.experimental.pallas import tpu_sc as plsc

N = 10000
E = 160000
D = 256
HALF = D // 2
NC = 2
NS = 16
CHUNK = 128
RPT = 640
NPAD = NS * RPT
UNROLL_B = 4
BULK = 8
EPAD = 163840
TURNS = EPAD // (CHUNK * BULK * NS)
EIDX = EPAD + 2 * NS * BULK * CHUNK


@functools.cache
def _sc_kernels():
    mesh = plsc.VectorSubcoreMesh(core_axis_name="c", subcore_axis_name="s")

    @functools.partial(
        pl.kernel,
        out_type=jax.ShapeDtypeStruct((NC, NPAD, 16), jnp.float32),
        mesh=mesh,
        scratch_types=[
            pltpu.VMEM((UNROLL_B, CHUNK), jnp.int32),
            pltpu.VMEM((CHUNK, 16), jnp.float32),
            pltpu.VMEM_SHARED((NPAD, 16), jnp.float32),
            pltpu.SemaphoreType.DMA((UNROLL_B,)),
        ],
    )
    def sc_bincount(src_hbm, zeros_hbm, ones_hbm, out_hbm, idxv, onesv, acc,
                    sem):
        c = lax.axis_index("c")
        s = lax.axis_index("s")
        row0 = s * RPT
        pltpu.sync_copy(zeros_hbm, acc.at[pl.ds(row0, RPT)])
        pltpu.sync_copy(ones_hbm, onesv)
        plsc.subcore_barrier()

        ehalf = EPAD // NC
        nch = ehalf // CHUNK
        iters = nch // (NS * UNROLL_B)

        @pl.loop(0, iters)
        def _(t):
            base = c * ehalf + (t * NS * UNROLL_B + s * UNROLL_B) * CHUNK
            for j in range(UNROLL_B):
                pltpu.async_copy(src_hbm.at[pl.ds(base + j * CHUNK, CHUNK)],
                                 idxv.at[j], sem.at[j])
            for j in range(UNROLL_B):
                pltpu.make_async_copy(src_hbm.at[pl.ds(base, CHUNK)],
                                      idxv.at[j], sem.at[j]).wait()
                pltpu.async_copy(onesv, acc.at[idxv.at[j]], sem.at[j],
                                 add=True)
            for j in range(UNROLL_B):
                pltpu.make_async_copy(onesv, acc.at[idxv.at[j]],
                                      sem.at[j]).wait()

        plsc.subcore_barrier()
        pltpu.sync_copy(acc.at[pl.ds(row0, RPT)],
                        out_hbm.at[c, pl.ds(row0, RPT)])

    @functools.partial(
        pl.kernel,
        out_type=jax.ShapeDtypeStruct((NC, NPAD, HALF), jnp.float32),
        mesh=mesh,
        scratch_types=[
            pltpu.VMEM((BULK, CHUNK), jnp.int32),
            pltpu.VMEM((BULK, CHUNK), jnp.int32),
            pltpu.VMEM((BULK, CHUNK), jnp.int32),
            pltpu.VMEM((BULK, CHUNK), jnp.int32),
            pltpu.VMEM((CHUNK, HALF), jnp.float32),
            pltpu.VMEM_SHARED((NPAD, HALF), jnp.float32),
            pltpu.SemaphoreType.DMA((2,)),
            pltpu.SemaphoreType.DMA,
        ],
    )
    def sc_scatter(g0_hbm, g1_hbm, src2_hbm, dst2_hbm, zeros_hbm, out_hbm,
                   srcb0, dstb0, srcb1, dstb1, gbuf, acc, isem, gsem):
        c = lax.axis_index("c")
        s = lax.axis_index("s")
        row0 = s * RPT
        pltpu.sync_copy(zeros_hbm, acc.at[pl.ds(row0, RPT)])
        plsc.subcore_barrier()

        def prefetch(turn, srcb, dstb, p):
            base = (turn * NS + s) * BULK
            pltpu.async_copy(src2_hbm.at[pl.ds(base, BULK)], srcb, isem.at[p])
            pltpu.async_copy(dst2_hbm.at[pl.ds(base, BULK)], dstb, isem.at[p])

        def wait_prefetch(srcb, dstb, p):
            pltpu.make_async_copy(src2_hbm.at[pl.ds(0, BULK)],
                                  srcb, isem.at[p]).wait()
            pltpu.make_async_copy(dst2_hbm.at[pl.ds(0, BULK)],
                                  dstb, isem.at[p]).wait()

        def run(g_hbm):
            prefetch(0, srcb0, dstb0, 0)
            prefetch(1, srcb1, dstb1, 1)

            @pl.loop(0, TURNS // 2)
            def _(u):
                wait_prefetch(srcb0, dstb0, 0)
                for j in range(BULK):
                    pltpu.async_copy(g_hbm.at[srcb0.at[j]], gbuf, gsem).wait()
                    pltpu.sync_copy(gbuf, acc.at[dstb0.at[j]], add=True)
                prefetch(2 * u + 2, srcb0, dstb0, 0)
                wait_prefetch(srcb1, dstb1, 1)
                for j in range(BULK):
                    pltpu.async_copy(g_hbm.at[srcb1.at[j]], gbuf, gsem).wait()
                    pltpu.sync_copy(gbuf, acc.at[dstb1.at[j]], add=True)
                prefetch(2 * u + 3, srcb1, dstb1, 1)

            wait_prefetch(srcb0, dstb0, 0)
            wait_prefetch(srcb1, dstb1, 1)

        @pl.when(c == 0)
        def _():
            run(g0_hbm)

        @pl.when(c == 1)
        def _():
            run(g1_hbm)

        plsc.subcore_barrier()
        pltpu.sync_copy(acc.at[pl.ds(row0, RPT)],
                        out_hbm.at[c, pl.ds(row0, RPT)])

    return sc_bincount, sc_scatter


_ROWS = 1000


def _tc_main_body(counts_ref, x_ref, w_ref, b_ref, root_ref,
                  g0_ref, g1_ref, self_ref):
    cnt = counts_ref[0, :, 0:1] + counts_ref[1, :, 0:1]
    degs = cnt + 1.0
    norm = lax.rsqrt(degs)
    h = jnp.dot(x_ref[...], w_ref[...],
                preferred_element_type=jnp.float32) + b_ref[...]
    g = norm * jnp.maximum(h, 0.0)
    g0_ref[...] = g[:, :HALF]
    g1_ref[...] = g[:, HALF:]
    self_ref[...] = jnp.maximum(h + root_ref[...], 0.0) / degs


def _tc_main(counts, x, w, b2, root):
    return pl.pallas_call(
        _tc_main_body,
        grid=(N // _ROWS,),
        in_specs=[
            pl.BlockSpec((NC, _ROWS, 16), lambda i: (0, i, 0)),
            pl.BlockSpec((_ROWS, D), lambda i: (i, 0)),
            pl.BlockSpec((D, D), lambda i: (0, 0)),
            pl.BlockSpec((1, D), lambda i: (0, 0)),
            pl.BlockSpec((1, D), lambda i: (0, 0)),
        ],
        out_specs=[
            pl.BlockSpec((_ROWS, HALF), lambda i: (i, 0)),
            pl.BlockSpec((_ROWS, HALF), lambda i: (i, 0)),
            pl.BlockSpec((_ROWS, D), lambda i: (i, 0)),
        ],
        out_shape=[
            jax.ShapeDtypeStruct((NPAD, HALF), jnp.float32),
            jax.ShapeDtypeStruct((NPAD, HALF), jnp.float32),
            jax.ShapeDtypeStruct((N, D), jnp.float32),
        ],
    )(counts, x, w, b2, root)


def _tc_out_body(counts_ref, acc_ref, self_ref, o_ref):
    cnt = counts_ref[0, :, 0:1] + counts_ref[1, :, 0:1]
    norm = lax.rsqrt(cnt + 1.0)
    acc = jnp.concatenate([acc_ref[0], acc_ref[1]], axis=1)
    o_ref[...] = norm * acc + self_ref[...]


def _tc_out(counts, acc, self_term):
    return pl.pallas_call(
        _tc_out_body,
        grid=(N // _ROWS,),
        in_specs=[
            pl.BlockSpec((NC, _ROWS, 16), lambda i: (0, i, 0)),
            pl.BlockSpec((NC, _ROWS, HALF), lambda i: (0, i, 0)),
            pl.BlockSpec((_ROWS, D), lambda i: (i, 0)),
        ],
        out_specs=pl.BlockSpec((_ROWS, D), lambda i: (i, 0)),
        out_shape=jax.ShapeDtypeStruct((N, D), jnp.float32),
    )(counts, acc, self_term)


def kernel(x, edge_index, W, b, root_emb):
    sc_bincount, sc_scatter = _sc_kernels()
    pad = jnp.full((EIDX - E,), N, jnp.int32)
    src = jnp.concatenate([edge_index[0], pad])
    dst = jnp.concatenate([edge_index[1], pad])
    zeros16 = jnp.zeros((RPT, 16), jnp.float32)
    ones16 = jnp.ones((CHUNK, 16), jnp.float32)
    zeros128 = jnp.zeros((RPT, HALF), jnp.float32)

    counts = sc_bincount(src, zeros16, ones16)
    g0, g1, self_term = _tc_main(counts, x, W, b.reshape(1, D), root_emb)
    acc = sc_scatter(g0, g1, src.reshape(-1, CHUNK), dst.reshape(-1, CHUNK),
                     zeros128)
    return _tc_out(counts, acc, self_term)
